# baseline (speedup 1.0000x reference)
import functools

import jax
import jax.numpy as jnp
from jax import lax
from jax.experimental import pallas as pl
from jax.experimental.pallas import tpu as pltpu
from jax.experimental.pallas import tpu_sc as plsc

N = 10000
E = 320000
D = 128

NC = 2
NS = 16
NW = NC * NS
EPW = E // NW
EC = 40
NCH = EPW // EC
NP = 10240
RPT = NP // NS
OC = 80

NBLK = 400
NGRID = N // NBLK
EBLK = 2000
EGRID = E // EBLK


def _sc_seg_body(a_hbm, b_hbm, c_hbm, src_hbm, dst_hbm, out_hbm,
                 srcv, dstv, av, bv, cv, zbuf, acc,
                 sidx, sga, sgb, sgc, ssc):
    cid = lax.axis_index("c")
    sid = lax.axis_index("s")
    wid = cid * NS + sid
    base0 = wid * EPW

    def idx_start(i4, qbase):
        pltpu.async_copy(src_hbm.at[pl.ds(qbase, EC)], srcv[i4], sidx[i4])
        pltpu.async_copy(dst_hbm.at[pl.ds(qbase, EC)], dstv[i4], sidx[i4])

    def idx_wait(i4, qbase):
        pltpu.make_async_copy(src_hbm.at[pl.ds(qbase, EC)], srcv[i4],
                              sidx[i4]).wait()
        pltpu.make_async_copy(dst_hbm.at[pl.ds(qbase, EC)], dstv[i4],
                              sidx[i4]).wait()

    def gathers_start(i4, p, qbase):
        pltpu.async_copy(a_hbm.at[srcv[i4]], av[p], sga[p])
        pltpu.async_copy(b_hbm.at[dstv[i4]], bv[p], sgb[p])
        pltpu.async_copy(c_hbm.at[pl.ds(qbase, EC)], cv[p], sgc[p])

    def gathers_wait(i4, p, qbase):
        pltpu.make_async_copy(a_hbm.at[srcv[i4]], av[p], sga[p]).wait()
        pltpu.make_async_copy(b_hbm.at[dstv[i4]], bv[p], sgb[p]).wait()
        pltpu.make_async_copy(c_hbm.at[pl.ds(qbase, EC)], cv[p], sgc[p]).wait()

    def scatter_wait(i4, p):
        pltpu.make_async_copy(cv[p], acc.at[srcv[i4]], ssc[p]).wait()

    idx_start(0, base0)
    idx_start(1, base0 + EC)
    idx_wait(0, base0)
    gathers_start(0, 0, base0)

    def zloop(e, carry):
        for j in range(8):
            zbuf[e, pl.ds(j * 16, 16)] = jnp.zeros((16,), jnp.float32)
        return carry

    lax.fori_loop(0, EC, zloop, 0)
    for k in range(RPT // EC):
        pltpu.sync_copy(zbuf, acc.at[pl.ds(sid * RPT + k * EC, EC)])
    plsc.subcore_barrier()

    def stage(t, carry):
        for r in range(4):
            q = 4 * t + r
            p = r % 2
            qb = base0 + q * EC

            @pl.when(jnp.logical_and(q >= 1, q < NCH))
            def _():
                scatter_wait((r - 1) % 4, (r - 1) % 2)

            @pl.when(q + 2 < NCH)
            def _():
                idx_start((r + 2) % 4, qb + 2 * EC)

            @pl.when(q + 1 < NCH)
            def _():
                idx_wait((r + 1) % 4, qb + EC)
                gathers_start((r + 1) % 4, (r + 1) % 2, qb + EC)

            @pl.when(q < NCH)
            def _():
                gathers_wait(r, p, qb)

                def compute(e, c2):
                    for j in range(8):
                        sl = pl.ds(j * 16, 16)
                        cv[p][e, sl] = jnp.maximum(
                            av[p][e, sl] + bv[p][e, sl] + cv[p][e, sl], 0.0)
                    return c2

                lax.fori_loop(0, EC, compute, 0)
                pltpu.async_copy(cv[p], acc.at[srcv[r]], ssc[p], add=True)
        return carry

    lax.fori_loop(0, (NCH + 3) // 4, stage, 0)
    scatter_wait((NCH - 1) % 4, (NCH - 1) % 2)
    plsc.subcore_barrier()
    for k in range(RPT // OC):
        r0 = sid * RPT + k * OC

        @pl.when(r0 < N)
        def _():
            pltpu.sync_copy(acc.at[pl.ds(r0, OC)],
                            out_hbm.at[cid, pl.ds(r0, OC)])


def _sc_segment_pass(a_tab, b_tab, c_tab, src, dst):
    mesh = plsc.VectorSubcoreMesh(core_axis_name="c", subcore_axis_name="s")
    f = pl.kernel(
        _sc_seg_body,
        out_type=jax.ShapeDtypeStruct((NC, N, D), jnp.float32),
        mesh=mesh,
        scratch_types=[
            [pltpu.VMEM((EC,), jnp.int32) for _ in range(4)],
            [pltpu.VMEM((EC,), jnp.int32) for _ in range(4)],
            [pltpu.VMEM((EC, D), jnp.float32) for _ in range(2)],
            [pltpu.VMEM((EC, D), jnp.float32) for _ in range(2)],
            [pltpu.VMEM((EC, D), jnp.float32) for _ in range(2)],
            pltpu.VMEM((EC, D), jnp.float32),
            pltpu.VMEM_SHARED((NP, D), jnp.float32),
            [pltpu.SemaphoreType.DMA for _ in range(4)],
            [pltpu.SemaphoreType.DMA for _ in range(2)],
            [pltpu.SemaphoreType.DMA for _ in range(2)],
            [pltpu.SemaphoreType.DMA for _ in range(2)],
            [pltpu.SemaphoreType.DMA for _ in range(2)],
        ],
    )
    return f(a_tab, b_tab, c_tab, src, dst)


def _nodes0_body(cf_ref, ex_ref, wc_ref, bc_ref, ws_ref, wd_ref,
                 a_ref, b_ref, psum_ref, esum_ref):
    i = pl.program_id(0)
    h = jnp.maximum(cf_ref[...] @ wc_ref[...] + bc_ref[...], 0.0) * ex_ref[...]
    a_ref[...] = h @ ws_ref[...]
    b_ref[...] = h @ wd_ref[...]

    @pl.when(i == 0)
    def _():
        psum_ref[...] = jnp.zeros_like(psum_ref)
        esum_ref[...] = jnp.zeros_like(esum_ref)

    psum_ref[...] += jnp.sum(h, axis=0, keepdims=True)
    esum_ref[...] += jnp.sum(ex_ref[...], axis=0, keepdims=True)


def _nodes0(cf, ex, w_child, bc, ws, wd):
    return pl.pallas_call(
        _nodes0_body,
        grid=(NGRID,),
        in_specs=[
            pl.BlockSpec((NBLK, D), lambda i: (i, 0)),
            pl.BlockSpec((NBLK, 1), lambda i: (i, 0)),
            pl.BlockSpec((D, D), lambda i: (0, 0)),
            pl.BlockSpec((1, D), lambda i: (0, 0)),
            pl.BlockSpec((D, D), lambda i: (0, 0)),
            pl.BlockSpec((D, D), lambda i: (0, 0)),
        ],
        out_specs=[
            pl.BlockSpec((NBLK, D), lambda i: (i, 0)),
            pl.BlockSpec((NBLK, D), lambda i: (i, 0)),
            pl.BlockSpec((1, D), lambda i: (0, 0)),
            pl.BlockSpec((1, 1), lambda i: (0, 0)),
        ],
        out_shape=[
            jax.ShapeDtypeStruct((N, D), jnp.float32),
            jax.ShapeDtypeStruct((N, D), jnp.float32),
            jax.ShapeDtypeStruct((1, D), jnp.float32),
            jax.ShapeDtypeStruct((1, 1), jnp.float32),
        ],
    )(cf, ex, w_child, bc, ws, wd)


def _edges_body(et_ref, efe_ref, wt_ref, wf_ref, b_ref, c_ref):
    c_ref[...] = (et_ref[...] @ wt_ref[...] + efe_ref[...] @ wf_ref[...]
                  + b_ref[...])


def _edges(et, efe, wt, wf, b):
    return pl.pallas_call(
        _edges_body,
        grid=(EGRID,),
        in_specs=[
            pl.BlockSpec((EBLK, 4), lambda i: (i, 0)),
            pl.BlockSpec((EBLK, 16), lambda i: (i, 0)),
            pl.BlockSpec((4, D), lambda i: (0, 0)),
            pl.BlockSpec((16, D), lambda i: (0, 0)),
            pl.BlockSpec((1, D), lambda i: (0, 0)),
        ],
        out_specs=pl.BlockSpec((EBLK, D), lambda i: (i, 0)),
        out_shape=jax.ShapeDtypeStruct((E, D), jnp.float32),
    )(et, efe, wt, wf, b)


def _nodes1_body(hp_ref, ws_ref, wd_ref, a_ref, b_ref, psum_ref):
    i = pl.program_id(0)
    h = hp_ref[0] + hp_ref[1]
    a_ref[...] = h @ ws_ref[...]
    b_ref[...] = h @ wd_ref[...]

    @pl.when(i == 0)
    def _():
        psum_ref[...] = jnp.zeros_like(psum_ref)

    psum_ref[...] += jnp.sum(h, axis=0, keepdims=True)


def _nodes1(hp, ws, wd):
    return pl.pallas_call(
        _nodes1_body,
        grid=(NGRID,),
        in_specs=[
            pl.BlockSpec((NC, NBLK, D), lambda i: (0, i, 0)),
            pl.BlockSpec((D, D), lambda i: (0, 0)),
            pl.BlockSpec((D, D), lambda i: (0, 0)),
        ],
        out_specs=[
            pl.BlockSpec((NBLK, D), lambda i: (i, 0)),
            pl.BlockSpec((NBLK, D), lambda i: (i, 0)),
            pl.BlockSpec((1, D), lambda i: (0, 0)),
        ],
        out_shape=[
            jax.ShapeDtypeStruct((N, D), jnp.float32),
            jax.ShapeDtypeStruct((N, D), jnp.float32),
            jax.ShapeDtypeStruct((1, D), jnp.float32),
        ],
    )(hp, ws, wd)


def _final_body(h2p_ref, p0_ref, p1_ref, esum_ref, wp0_ref, wp1_ref, wp2_ref,
                bp_ref, out_ref, p2_acc):
    i = pl.program_id(0)

    @pl.when(i == 0)
    def _():
        p2_acc[...] = jnp.zeros_like(p2_acc)

    p2_acc[...] += (jnp.sum(h2p_ref[0], axis=0, keepdims=True)
                    + jnp.sum(h2p_ref[1], axis=0, keepdims=True))

    @pl.when(i == NGRID - 1)
    def _():
        s = (p0_ref[...] @ wp0_ref[...]
             + p1_ref[...] @ wp1_ref[...]
             + p2_acc[...] @ wp2_ref[...]) / esum_ref[...] + bp_ref[...]
        out_ref[...] = jnp.maximum(s, 0.0)


def _final(h2p, p0, p1, esum, wp0, wp1, wp2, bp):
    return pl.pallas_call(
        _final_body,
        grid=(NGRID,),
        in_specs=[
            pl.BlockSpec((NC, NBLK, D), lambda i: (0, i, 0)),
            pl.BlockSpec((1, D), lambda i: (0, 0)),
            pl.BlockSpec((1, D), lambda i: (0, 0)),
            pl.BlockSpec((1, 1), lambda i: (0, 0)),
            pl.BlockSpec((D, D), lambda i: (0, 0)),
            pl.BlockSpec((D, D), lambda i: (0, 0)),
            pl.BlockSpec((D, D), lambda i: (0, 0)),
            pl.BlockSpec((1, D), lambda i: (0, 0)),
        ],
        out_specs=pl.BlockSpec((1, D), lambda i: (0, 0)),
        out_shape=jax.ShapeDtypeStruct((1, D), jnp.float32),
        scratch_shapes=[pltpu.VMEM((1, D), jnp.float32)],
    )(h2p, p0, p1, esum, wp0, wp1, wp2, bp)


def kernel(child_feats, child_exists, edge_type_onehot, edge_feats,
           edge_indices, W_child, b_child, W_ne0, b_ne0, W_ne1, b_ne1,
           W_parent, b_parent):
    cf = child_feats[0]
    ex = child_exists[0]
    et = edge_type_onehot[0]
    efe = edge_feats[0]
    src = edge_indices[0, :, 0].astype(jnp.int32)
    dst = edge_indices[0, :, 1].astype(jnp.int32)

    ws0, wd0 = W_ne0[:D], W_ne0[D:2 * D]
    wt0, wf0 = W_ne0[2 * D:2 * D + 4], W_ne0[2 * D + 4:]
    ws1, wd1 = W_ne1[:D], W_ne1[D:2 * D]
    wt1, wf1 = W_ne1[2 * D:2 * D + 4], W_ne1[2 * D + 4:]
    b0 = b_ne0.reshape(1, D)
    b1 = b_ne1.reshape(1, D)
    bc = b_child.reshape(1, D)
    wp0, wp1, wp2 = W_parent[:D], W_parent[D:2 * D], W_parent[2 * D:]
    bp = b_parent.reshape(1, D)

    a0, b0t, p0, esum = _nodes0(cf, ex, W_child, bc, ws0, wd0)
    c0 = _edges(et, efe, wt0, wf0, b0)
    c1 = _edges(et, efe, wt1, wf1, b1)
    h1p = _sc_segment_pass(a0, b0t, c0, src, dst)
    a1, b1t, p1 = _nodes1(h1p, ws1, wd1)
    h2p = _sc_segment_pass(a1, b1t, c1, src, dst)
    return _final(h2p, p0, p1, esum, wp0, wp1, wp2, bp)


# speedup vs baseline: 1.2708x; 1.2708x over previous
import functools

import jax
import jax.numpy as jnp
from jax import lax
from jax.experimental import pallas as pl
from jax.experimental.pallas import tpu as pltpu
from jax.experimental.pallas import tpu_sc as plsc

N = 10000
E = 320000
D = 128

NC = 2
NS = 16
NW = NC * NS
EPW = E // NW
EC = 40
NCH = EPW // EC
NP = 10240
RPT = NP // NS
OC = 80

NBLK = 400
NGRID = N // NBLK
EBLK = 3200
EGRID = E // EBLK


def _sc_seg_body(a_hbm, b_hbm, c_hbm, src_hbm, dst_hbm, out_hbm,
                 srcv, dstv, av, bv, cv, zbuf, acc,
                 sidx, sga, sgb, sgc, ssc):
    cid = lax.axis_index("c")
    sid = lax.axis_index("s")
    wid = cid * NS + sid
    base0 = wid * EPW

    def idx_start(i4, qbase):
        pltpu.async_copy(src_hbm.at[pl.ds(qbase, EC)], srcv[i4], sidx[i4])
        pltpu.async_copy(dst_hbm.at[pl.ds(qbase, EC)], dstv[i4], sidx[i4])

    def idx_wait(i4, qbase):
        pltpu.make_async_copy(src_hbm.at[pl.ds(qbase, EC)], srcv[i4],
                              sidx[i4]).wait()
        pltpu.make_async_copy(dst_hbm.at[pl.ds(qbase, EC)], dstv[i4],
                              sidx[i4]).wait()

    def gathers_start(i4, p, qbase):
        pltpu.async_copy(a_hbm.at[srcv[i4]], av[p], sga[p])
        pltpu.async_copy(b_hbm.at[dstv[i4]], bv[p], sgb[p])
        pltpu.async_copy(c_hbm.at[pl.ds(qbase, EC)], cv[p], sgc[p])

    def gathers_wait(i4, p, qbase):
        pltpu.make_async_copy(a_hbm.at[srcv[i4]], av[p], sga[p]).wait()
        pltpu.make_async_copy(b_hbm.at[dstv[i4]], bv[p], sgb[p]).wait()
        pltpu.make_async_copy(c_hbm.at[pl.ds(qbase, EC)], cv[p], sgc[p]).wait()

    def scatter_wait(i4, p):
        pltpu.make_async_copy(cv[p], acc.at[srcv[i4]], ssc[p]).wait()

    idx_start(0, base0)
    idx_start(1, base0 + EC)
    idx_wait(0, base0)
    gathers_start(0, 0, base0)

    def zloop(e, carry):
        for j in range(8):
            zbuf[e, pl.ds(j * 16, 16)] = jnp.zeros((16,), jnp.float32)
        return carry

    lax.fori_loop(0, EC, zloop, 0)
    for k in range(RPT // EC):
        pltpu.sync_copy(zbuf, acc.at[pl.ds(sid * RPT + k * EC, EC)])
    plsc.subcore_barrier()

    def stage(t, carry):
        for r in range(4):
            q = 4 * t + r
            p = r % 2
            qb = base0 + q * EC

            @pl.when(jnp.logical_and(q >= 1, q < NCH))
            def _():
                scatter_wait((r - 1) % 4, (r - 1) % 2)

            @pl.when(q + 2 < NCH)
            def _():
                idx_start((r + 2) % 4, qb + 2 * EC)

            @pl.when(q + 1 < NCH)
            def _():
                idx_wait((r + 1) % 4, qb + EC)
                gathers_start((r + 1) % 4, (r + 1) % 2, qb + EC)

            @pl.when(q < NCH)
            def _():
                gathers_wait(r, p, qb)

                def compute(e, c2):
                    for j in range(8):
                        sl = pl.ds(j * 16, 16)
                        cv[p][e, sl] = jnp.maximum(
                            av[p][e, sl] + bv[p][e, sl] + cv[p][e, sl], 0.0)
                    return c2

                lax.fori_loop(0, EC, compute, 0)
                pltpu.async_copy(cv[p], acc.at[srcv[r]], ssc[p], add=True)
        return carry

    lax.fori_loop(0, (NCH + 3) // 4, stage, 0)
    scatter_wait((NCH - 1) % 4, (NCH - 1) % 2)
    plsc.subcore_barrier()
    for k in range(RPT // OC):
        r0 = sid * RPT + k * OC

        @pl.when(r0 < N)
        def _():
            pltpu.sync_copy(acc.at[pl.ds(r0, OC)],
                            out_hbm.at[cid, pl.ds(r0, OC)])


def _sc_segment_pass(a_tab, b_tab, c_tab, src, dst):
    mesh = plsc.VectorSubcoreMesh(core_axis_name="c", subcore_axis_name="s")
    f = pl.kernel(
        _sc_seg_body,
        out_type=jax.ShapeDtypeStruct((NC, N, D), jnp.float32),
        mesh=mesh,
        scratch_types=[
            [pltpu.VMEM((EC,), jnp.int32) for _ in range(4)],
            [pltpu.VMEM((EC,), jnp.int32) for _ in range(4)],
            [pltpu.VMEM((EC, D), jnp.float32) for _ in range(2)],
            [pltpu.VMEM((EC, D), jnp.float32) for _ in range(2)],
            [pltpu.VMEM((EC, D), jnp.float32) for _ in range(2)],
            pltpu.VMEM((EC, D), jnp.float32),
            pltpu.VMEM_SHARED((NP, D), jnp.float32),
            [pltpu.SemaphoreType.DMA for _ in range(4)],
            [pltpu.SemaphoreType.DMA for _ in range(2)],
            [pltpu.SemaphoreType.DMA for _ in range(2)],
            [pltpu.SemaphoreType.DMA for _ in range(2)],
            [pltpu.SemaphoreType.DMA for _ in range(2)],
        ],
    )
    return f(a_tab, b_tab, c_tab, src, dst)


def _nodes0_body(cf_ref, ex_ref, wc_ref, bc_ref, ws_ref, wd_ref,
                 a_ref, b_ref, psum_ref, esum_ref):
    i = pl.program_id(0)
    h = jnp.maximum(cf_ref[...] @ wc_ref[...] + bc_ref[...], 0.0) * ex_ref[...]
    a_ref[...] = h @ ws_ref[...]
    b_ref[...] = h @ wd_ref[...]

    @pl.when(i == 0)
    def _():
        psum_ref[...] = jnp.zeros_like(psum_ref)
        esum_ref[...] = jnp.zeros_like(esum_ref)

    psum_ref[...] += jnp.sum(h, axis=0, keepdims=True)
    esum_ref[...] += jnp.sum(ex_ref[...], axis=0, keepdims=True)


def _nodes0(cf, ex, w_child, bc, ws, wd):
    return pl.pallas_call(
        _nodes0_body,
        grid=(NGRID,),
        in_specs=[
            pl.BlockSpec((NBLK, D), lambda i: (i, 0)),
            pl.BlockSpec((NBLK, 1), lambda i: (i, 0)),
            pl.BlockSpec((D, D), lambda i: (0, 0)),
            pl.BlockSpec((1, D), lambda i: (0, 0)),
            pl.BlockSpec((D, D), lambda i: (0, 0)),
            pl.BlockSpec((D, D), lambda i: (0, 0)),
        ],
        out_specs=[
            pl.BlockSpec((NBLK, D), lambda i: (i, 0)),
            pl.BlockSpec((NBLK, D), lambda i: (i, 0)),
            pl.BlockSpec((1, D), lambda i: (0, 0)),
            pl.BlockSpec((1, 1), lambda i: (0, 0)),
        ],
        out_shape=[
            jax.ShapeDtypeStruct((N, D), jnp.float32),
            jax.ShapeDtypeStruct((N, D), jnp.float32),
            jax.ShapeDtypeStruct((1, D), jnp.float32),
            jax.ShapeDtypeStruct((1, 1), jnp.float32),
        ],
    )(cf, ex, w_child, bc, ws, wd)


def _edges_body(et_ref, efe_ref, wt_ref, wf_ref, b_ref, c_ref):
    dn = (((0,), (0,)), ((), ()))
    c_ref[...] = (lax.dot_general(et_ref[...], wt_ref[...], dn)
                  + lax.dot_general(efe_ref[...], wf_ref[...], dn)
                  + b_ref[...])


def _edges(ett, eft, wt, wf, b):
    return pl.pallas_call(
        _edges_body,
        grid=(EGRID,),
        in_specs=[
            pl.BlockSpec((4, EBLK), lambda i: (0, i)),
            pl.BlockSpec((16, EBLK), lambda i: (0, i)),
            pl.BlockSpec((4, D), lambda i: (0, 0)),
            pl.BlockSpec((16, D), lambda i: (0, 0)),
            pl.BlockSpec((1, D), lambda i: (0, 0)),
        ],
        out_specs=pl.BlockSpec((EBLK, D), lambda i: (i, 0)),
        out_shape=jax.ShapeDtypeStruct((E, D), jnp.float32),
    )(ett, eft, wt, wf, b)


def _nodes1_body(hp_ref, ws_ref, wd_ref, a_ref, b_ref, psum_ref):
    i = pl.program_id(0)
    h = hp_ref[0] + hp_ref[1]
    a_ref[...] = h @ ws_ref[...]
    b_ref[...] = h @ wd_ref[...]

    @pl.when(i == 0)
    def _():
        psum_ref[...] = jnp.zeros_like(psum_ref)

    psum_ref[...] += jnp.sum(h, axis=0, keepdims=True)


def _nodes1(hp, ws, wd):
    return pl.pallas_call(
        _nodes1_body,
        grid=(NGRID,),
        in_specs=[
            pl.BlockSpec((NC, NBLK, D), lambda i: (0, i, 0)),
            pl.BlockSpec((D, D), lambda i: (0, 0)),
            pl.BlockSpec((D, D), lambda i: (0, 0)),
        ],
        out_specs=[
            pl.BlockSpec((NBLK, D), lambda i: (i, 0)),
            pl.BlockSpec((NBLK, D), lambda i: (i, 0)),
            pl.BlockSpec((1, D), lambda i: (0, 0)),
        ],
        out_shape=[
            jax.ShapeDtypeStruct((N, D), jnp.float32),
            jax.ShapeDtypeStruct((N, D), jnp.float32),
            jax.ShapeDtypeStruct((1, D), jnp.float32),
        ],
    )(hp, ws, wd)


def _final_body(h2p_ref, p0_ref, p1_ref, esum_ref, wp0_ref, wp1_ref, wp2_ref,
                bp_ref, out_ref, p2_acc):
    i = pl.program_id(0)

    @pl.when(i == 0)
    def _():
        p2_acc[...] = jnp.zeros_like(p2_acc)

    p2_acc[...] += (jnp.sum(h2p_ref[0], axis=0, keepdims=True)
                    + jnp.sum(h2p_ref[1], axis=0, keepdims=True))

    @pl.when(i == NGRID - 1)
    def _():
        s = (p0_ref[...] @ wp0_ref[...]
             + p1_ref[...] @ wp1_ref[...]
             + p2_acc[...] @ wp2_ref[...]) / esum_ref[...] + bp_ref[...]
        out_ref[...] = jnp.maximum(s, 0.0)


def _final(h2p, p0, p1, esum, wp0, wp1, wp2, bp):
    return pl.pallas_call(
        _final_body,
        grid=(NGRID,),
        in_specs=[
            pl.BlockSpec((NC, NBLK, D), lambda i: (0, i, 0)),
            pl.BlockSpec((1, D), lambda i: (0, 0)),
            pl.BlockSpec((1, D), lambda i: (0, 0)),
            pl.BlockSpec((1, 1), lambda i: (0, 0)),
            pl.BlockSpec((D, D), lambda i: (0, 0)),
            pl.BlockSpec((D, D), lambda i: (0, 0)),
            pl.BlockSpec((D, D), lambda i: (0, 0)),
            pl.BlockSpec((1, D), lambda i: (0, 0)),
        ],
        out_specs=pl.BlockSpec((1, D), lambda i: (0, 0)),
        out_shape=jax.ShapeDtypeStruct((1, D), jnp.float32),
        scratch_shapes=[pltpu.VMEM((1, D), jnp.float32)],
    )(h2p, p0, p1, esum, wp0, wp1, wp2, bp)


def kernel(child_feats, child_exists, edge_type_onehot, edge_feats,
           edge_indices, W_child, b_child, W_ne0, b_ne0, W_ne1, b_ne1,
           W_parent, b_parent):
    cf = child_feats[0]
    ex = child_exists[0]
    ett = edge_type_onehot[0].T
    eft = edge_feats[0].T
    eit = edge_indices[0].T.astype(jnp.int32)
    src = eit[0]
    dst = eit[1]

    ws0, wd0 = W_ne0[:D], W_ne0[D:2 * D]
    wt0, wf0 = W_ne0[2 * D:2 * D + 4], W_ne0[2 * D + 4:]
    ws1, wd1 = W_ne1[:D], W_ne1[D:2 * D]
    wt1, wf1 = W_ne1[2 * D:2 * D + 4], W_ne1[2 * D + 4:]
    b0 = b_ne0.reshape(1, D)
    b1 = b_ne1.reshape(1, D)
    bc = b_child.reshape(1, D)
    wp0, wp1, wp2 = W_parent[:D], W_parent[D:2 * D], W_parent[2 * D:]
    bp = b_parent.reshape(1, D)

    a0, b0t, p0, esum = _nodes0(cf, ex, W_child, bc, ws0, wd0)
    c0 = _edges(ett, eft, wt0, wf0, b0)
    c1 = _edges(ett, eft, wt1, wf1, b1)
    h1p = _sc_segment_pass(a0, b0t, c0, src, dst)
    a1, b1t, p1 = _nodes1(h1p, ws1, wd1)
    h2p = _sc_segment_pass(a1, b1t, c1, src, dst)
    return _final(h2p, p0, p1, esum, wp0, wp1, wp2, bp)


# speedup vs baseline: 1.3592x; 1.0696x over previous
import functools

import jax
import jax.numpy as jnp
from jax import lax
from jax.experimental import pallas as pl
from jax.experimental.pallas import tpu as pltpu
from jax.experimental.pallas import tpu_sc as plsc

N = 10000
E = 320000
D = 128

NC = 2
NS = 16
NW = NC * NS
EPW = E // NW
EC = 40
NCH = EPW // EC
NP = 10240
RPT = NP // NS
OC = 80

NBLK = 400
NGRID = N // NBLK
EBLK = 3200
EGRID = E // EBLK


def _sc_seg_body(a_hbm, b_hbm, c_hbm, src_hbm, dst_hbm, out_hbm,
                 srcv, dstv, av, bv, cv, xv, zbuf, acc,
                 sidx, sga, sgb, sgc, ssc):
    cid = lax.axis_index("c")
    sid = lax.axis_index("s")
    wid = cid * NS + sid
    base0 = wid * EPW

    def idx_start(i4, qbase):
        pltpu.async_copy(src_hbm.at[pl.ds(qbase, EC)], srcv[i4], sidx[i4])
        pltpu.async_copy(dst_hbm.at[pl.ds(qbase, EC)], dstv[i4], sidx[i4])

    def idx_wait(i4, qbase):
        pltpu.make_async_copy(src_hbm.at[pl.ds(qbase, EC)], srcv[i4],
                              sidx[i4]).wait()
        pltpu.make_async_copy(dst_hbm.at[pl.ds(qbase, EC)], dstv[i4],
                              sidx[i4]).wait()

    def gathers_start(i4, p, qbase):
        pltpu.async_copy(a_hbm.at[srcv[i4]], av[p], sga[p])
        pltpu.async_copy(b_hbm.at[dstv[i4]], bv[p], sgb[p])
        pltpu.async_copy(c_hbm.at[pl.ds(qbase, EC)], cv[p], sgc[p])

    def gathers_wait(i4, p, qbase):
        pltpu.make_async_copy(a_hbm.at[srcv[i4]], av[p], sga[p]).wait()
        pltpu.make_async_copy(b_hbm.at[dstv[i4]], bv[p], sgb[p]).wait()
        pltpu.make_async_copy(c_hbm.at[pl.ds(qbase, EC)], cv[p], sgc[p]).wait()

    def scatter_wait(i4, p):
        pltpu.make_async_copy(xv[p], acc.at[srcv[i4]], ssc[p]).wait()

    idx_start(0, base0)
    idx_start(1, base0 + EC)
    idx_wait(0, base0)
    gathers_start(0, 0, base0)

    def zloop(e, carry):
        for j in range(8):
            zbuf[e, pl.ds(j * 16, 16)] = jnp.zeros((16,), jnp.float32)
        return carry

    lax.fori_loop(0, EC, zloop, 0)
    for k in range(RPT // EC):
        pltpu.sync_copy(zbuf, acc.at[pl.ds(sid * RPT + k * EC, EC)])
    plsc.subcore_barrier()

    def stage(t, carry):
        for r in range(4):
            q = 4 * t + r
            p = r % 2
            qb = base0 + q * EC

            @pl.when(jnp.logical_and(q >= 1, q < NCH))
            def _():
                scatter_wait((r - 1) % 4, (r - 1) % 2)

            @pl.when(q + 2 < NCH)
            def _():
                idx_start((r + 2) % 4, qb + 2 * EC)

            @pl.when(q + 1 < NCH)
            def _():
                idx_wait((r + 1) % 4, qb + EC)
                gathers_start((r + 1) % 4, (r + 1) % 2, qb + EC)

            @pl.when(q < NCH)
            def _():
                gathers_wait(r, p, qb)

                def compute(e, c2):
                    for j in range(8):
                        sl = pl.ds(j * 16, 16)
                        xv[p][e, sl] = jnp.maximum(
                            av[p][e, sl] + bv[p][e, sl] + cv[p][e, sl], 0.0)
                    return c2

                lax.fori_loop(0, EC, compute, 0)
                pltpu.async_copy(xv[p], acc.at[srcv[r]], ssc[p], add=True)
        return carry

    lax.fori_loop(0, (NCH + 3) // 4, stage, 0)
    scatter_wait((NCH - 1) % 4, (NCH - 1) % 2)
    plsc.subcore_barrier()
    for k in range(RPT // OC):
        r0 = sid * RPT + k * OC

        @pl.when(r0 < N)
        def _():
            pltpu.sync_copy(acc.at[pl.ds(r0, OC)],
                            out_hbm.at[cid, pl.ds(r0, OC)])


def _sc_segment_pass(a_tab, b_tab, c_tab, src, dst):
    mesh = plsc.VectorSubcoreMesh(core_axis_name="c", subcore_axis_name="s")
    f = pl.kernel(
        _sc_seg_body,
        out_type=jax.ShapeDtypeStruct((NC, N, D), jnp.float32),
        mesh=mesh,
        scratch_types=[
            [pltpu.VMEM((EC,), jnp.int32) for _ in range(4)],
            [pltpu.VMEM((EC,), jnp.int32) for _ in range(4)],
            [pltpu.VMEM((EC, D), jnp.float32) for _ in range(2)],
            [pltpu.VMEM((EC, D), jnp.float32) for _ in range(2)],
            [pltpu.VMEM((EC, D), jnp.float32) for _ in range(2)],
            [pltpu.VMEM((EC, D), jnp.float32) for _ in range(2)],
            pltpu.VMEM((EC, D), jnp.float32),
            pltpu.VMEM_SHARED((NP, D), jnp.float32),
            [pltpu.SemaphoreType.DMA for _ in range(4)],
            [pltpu.SemaphoreType.DMA for _ in range(2)],
            [pltpu.SemaphoreType.DMA for _ in range(2)],
            [pltpu.SemaphoreType.DMA for _ in range(2)],
            [pltpu.SemaphoreType.DMA for _ in range(2)],
        ],
    )
    return f(a_tab, b_tab, c_tab, src, dst)


def _sc_sum_body(a_hbm, b_hbm, c_hbm, src_hbm, dst_hbm, out_hbm,
                 srcv, dstv, av, bv, cv, accb, sidx, sga, sgb, sgc):
    cid = lax.axis_index("c")
    sid = lax.axis_index("s")
    wid = cid * NS + sid
    base0 = wid * EPW

    def idx_start(i4, qbase):
        pltpu.async_copy(src_hbm.at[pl.ds(qbase, EC)], srcv[i4], sidx[i4])
        pltpu.async_copy(dst_hbm.at[pl.ds(qbase, EC)], dstv[i4], sidx[i4])

    def idx_wait(i4, qbase):
        pltpu.make_async_copy(src_hbm.at[pl.ds(qbase, EC)], srcv[i4],
                              sidx[i4]).wait()
        pltpu.make_async_copy(dst_hbm.at[pl.ds(qbase, EC)], dstv[i4],
                              sidx[i4]).wait()

    def gathers_start(i4, p, qbase):
        pltpu.async_copy(a_hbm.at[srcv[i4]], av[p], sga[p])
        pltpu.async_copy(b_hbm.at[dstv[i4]], bv[p], sgb[p])
        pltpu.async_copy(c_hbm.at[pl.ds(qbase, EC)], cv[p], sgc[p])

    def gathers_wait(i4, p, qbase):
        pltpu.make_async_copy(a_hbm.at[srcv[i4]], av[p], sga[p]).wait()
        pltpu.make_async_copy(b_hbm.at[dstv[i4]], bv[p], sgb[p]).wait()
        pltpu.make_async_copy(c_hbm.at[pl.ds(qbase, EC)], cv[p], sgc[p]).wait()

    def chunk_sum(p, acc):
        def compute(e, a8):
            out = []
            for j in range(8):
                sl = pl.ds(j * 16, 16)
                x = jnp.maximum(
                    av[p][e, sl] + bv[p][e, sl] + cv[p][e, sl], 0.0)
                out.append(a8[j] + x)
            return tuple(out)

        return lax.fori_loop(0, EC, compute, acc)

    idx_start(0, base0)
    idx_start(1, base0 + EC)
    idx_wait(0, base0)
    gathers_start(0, 0, base0)
    zero = jnp.zeros((16,), jnp.float32)
    acc0 = (zero,) * 8

    def stage(t, acc):
        for r in range(4):
            q = 4 * t + r
            p = r % 2
            qb = base0 + q * EC
            idx_start((r + 2) % 4, qb + 2 * EC)
            idx_wait((r + 1) % 4, qb + EC)
            gathers_start((r + 1) % 4, (r + 1) % 2, qb + EC)
            gathers_wait(r, p, qb)
            acc = chunk_sum(p, acc)
        return acc

    acc = lax.fori_loop(0, (NCH - 2) // 4, stage, acc0)
    qb = base0 + (NCH - 2) * EC
    idx_wait((NCH - 1) % 4, qb + EC)
    gathers_start((NCH - 1) % 4, 1, qb + EC)
    gathers_wait((NCH - 2) % 4, 0, qb)
    acc = chunk_sum(0, acc)
    gathers_wait((NCH - 1) % 4, 1, qb + EC)
    acc = chunk_sum(1, acc)
    for j in range(8):
        accb[0, pl.ds(j * 16, 16)] = acc[j]
    pltpu.sync_copy(accb, out_hbm.at[wid])


def _sc_sum_pass(a_tab, b_tab, c_tab, src, dst):
    mesh = plsc.VectorSubcoreMesh(core_axis_name="c", subcore_axis_name="s")
    f = pl.kernel(
        _sc_sum_body,
        out_type=jax.ShapeDtypeStruct((NW, 8, D), jnp.float32),
        mesh=mesh,
        scratch_types=[
            [pltpu.VMEM((EC,), jnp.int32) for _ in range(4)],
            [pltpu.VMEM((EC,), jnp.int32) for _ in range(4)],
            [pltpu.VMEM((EC, D), jnp.float32) for _ in range(2)],
            [pltpu.VMEM((EC, D), jnp.float32) for _ in range(2)],
            [pltpu.VMEM((EC, D), jnp.float32) for _ in range(2)],
            pltpu.VMEM((8, D), jnp.float32),
            [pltpu.SemaphoreType.DMA for _ in range(4)],
            [pltpu.SemaphoreType.DMA for _ in range(2)],
            [pltpu.SemaphoreType.DMA for _ in range(2)],
            [pltpu.SemaphoreType.DMA for _ in range(2)],
        ],
    )
    return f(a_tab, b_tab, c_tab, src, dst)


def _nodes0_body(cf_ref, ex_ref, wc_ref, bc_ref, ws_ref, wd_ref,
                 a_ref, b_ref, psum_ref, esum_ref):
    i = pl.program_id(0)
    h = jnp.maximum(cf_ref[...] @ wc_ref[...] + bc_ref[...], 0.0) * ex_ref[...]
    a_ref[...] = h @ ws_ref[...]
    b_ref[...] = h @ wd_ref[...]

    @pl.when(i == 0)
    def _():
        psum_ref[...] = jnp.zeros_like(psum_ref)
        esum_ref[...] = jnp.zeros_like(esum_ref)

    psum_ref[...] += jnp.sum(h, axis=0, keepdims=True)
    esum_ref[...] += jnp.sum(ex_ref[...], axis=0, keepdims=True)


def _nodes0(cf, ex, w_child, bc, ws, wd):
    return pl.pallas_call(
        _nodes0_body,
        grid=(NGRID,),
        in_specs=[
            pl.BlockSpec((NBLK, D), lambda i: (i, 0)),
            pl.BlockSpec((NBLK, 1), lambda i: (i, 0)),
            pl.BlockSpec((D, D), lambda i: (0, 0)),
            pl.BlockSpec((1, D), lambda i: (0, 0)),
            pl.BlockSpec((D, D), lambda i: (0, 0)),
            pl.BlockSpec((D, D), lambda i: (0, 0)),
        ],
        out_specs=[
            pl.BlockSpec((NBLK, D), lambda i: (i, 0)),
            pl.BlockSpec((NBLK, D), lambda i: (i, 0)),
            pl.BlockSpec((1, D), lambda i: (0, 0)),
            pl.BlockSpec((1, 1), lambda i: (0, 0)),
        ],
        out_shape=[
            jax.ShapeDtypeStruct((N, D), jnp.float32),
            jax.ShapeDtypeStruct((N, D), jnp.float32),
            jax.ShapeDtypeStruct((1, D), jnp.float32),
            jax.ShapeDtypeStruct((1, 1), jnp.float32),
        ],
    )(cf, ex, w_child, bc, ws, wd)


def _edges_body(et_ref, efe_ref, wt_ref, wf_ref, b_ref, c_ref):
    dn = (((0,), (0,)), ((), ()))
    c_ref[...] = (lax.dot_general(et_ref[...], wt_ref[...], dn)
                  + lax.dot_general(efe_ref[...], wf_ref[...], dn)
                  + b_ref[...])


def _edges(ett, eft, wt, wf, b):
    return pl.pallas_call(
        _edges_body,
        grid=(EGRID,),
        in_specs=[
            pl.BlockSpec((4, EBLK), lambda i: (0, i)),
            pl.BlockSpec((16, EBLK), lambda i: (0, i)),
            pl.BlockSpec((4, D), lambda i: (0, 0)),
            pl.BlockSpec((16, D), lambda i: (0, 0)),
            pl.BlockSpec((1, D), lambda i: (0, 0)),
        ],
        out_specs=pl.BlockSpec((EBLK, D), lambda i: (i, 0)),
        out_shape=jax.ShapeDtypeStruct((E, D), jnp.float32),
    )(ett, eft, wt, wf, b)


def _nodes1_body(hp_ref, ws_ref, wd_ref, a_ref, b_ref, psum_ref):
    i = pl.program_id(0)
    h = hp_ref[0] + hp_ref[1]
    a_ref[...] = h @ ws_ref[...]
    b_ref[...] = h @ wd_ref[...]

    @pl.when(i == 0)
    def _():
        psum_ref[...] = jnp.zeros_like(psum_ref)

    psum_ref[...] += jnp.sum(h, axis=0, keepdims=True)


def _nodes1(hp, ws, wd):
    return pl.pallas_call(
        _nodes1_body,
        grid=(NGRID,),
        in_specs=[
            pl.BlockSpec((NC, NBLK, D), lambda i: (0, i, 0)),
            pl.BlockSpec((D, D), lambda i: (0, 0)),
            pl.BlockSpec((D, D), lambda i: (0, 0)),
        ],
        out_specs=[
            pl.BlockSpec((NBLK, D), lambda i: (i, 0)),
            pl.BlockSpec((NBLK, D), lambda i: (i, 0)),
            pl.BlockSpec((1, D), lambda i: (0, 0)),
        ],
        out_shape=[
            jax.ShapeDtypeStruct((N, D), jnp.float32),
            jax.ShapeDtypeStruct((N, D), jnp.float32),
            jax.ShapeDtypeStruct((1, D), jnp.float32),
        ],
    )(hp, ws, wd)


def _final_body(p2r_ref, p0_ref, p1_ref, esum_ref, wp0_ref, wp1_ref, wp2_ref,
                bp_ref, out_ref):
    p2 = jnp.sum(p2r_ref[:, 0, :], axis=0, keepdims=True)
    s = (p0_ref[...] @ wp0_ref[...]
         + p1_ref[...] @ wp1_ref[...]
         + p2 @ wp2_ref[...]) / esum_ref[...] + bp_ref[...]
    out_ref[...] = jnp.maximum(s, 0.0)


def _final(p2r, p0, p1, esum, wp0, wp1, wp2, bp):
    return pl.pallas_call(
        _final_body,
        grid=(1,),
        in_specs=[
            pl.BlockSpec((NW, 8, D), lambda i: (0, 0, 0)),
            pl.BlockSpec((1, D), lambda i: (0, 0)),
            pl.BlockSpec((1, D), lambda i: (0, 0)),
            pl.BlockSpec((1, 1), lambda i: (0, 0)),
            pl.BlockSpec((D, D), lambda i: (0, 0)),
            pl.BlockSpec((D, D), lambda i: (0, 0)),
            pl.BlockSpec((D, D), lambda i: (0, 0)),
            pl.BlockSpec((1, D), lambda i: (0, 0)),
        ],
        out_specs=pl.BlockSpec((1, D), lambda i: (0, 0)),
        out_shape=jax.ShapeDtypeStruct((1, D), jnp.float32),
    )(p2r, p0, p1, esum, wp0, wp1, wp2, bp)


def kernel(child_feats, child_exists, edge_type_onehot, edge_feats,
           edge_indices, W_child, b_child, W_ne0, b_ne0, W_ne1, b_ne1,
           W_parent, b_parent):
    cf = child_feats[0]
    ex = child_exists[0]
    ett = edge_type_onehot[0].T
    eft = edge_feats[0].T
    eit = edge_indices[0].T.astype(jnp.int32)
    src = eit[0]
    dst = eit[1]

    ws0, wd0 = W_ne0[:D], W_ne0[D:2 * D]
    wt0, wf0 = W_ne0[2 * D:2 * D + 4], W_ne0[2 * D + 4:]
    ws1, wd1 = W_ne1[:D], W_ne1[D:2 * D]
    wt1, wf1 = W_ne1[2 * D:2 * D + 4], W_ne1[2 * D + 4:]
    b0 = b_ne0.reshape(1, D)
    b1 = b_ne1.reshape(1, D)
    bc = b_child.reshape(1, D)
    wp0, wp1, wp2 = W_parent[:D], W_parent[D:2 * D], W_parent[2 * D:]
    bp = b_parent.reshape(1, D)

    a0, b0t, p0, esum = _nodes0(cf, ex, W_child, bc, ws0, wd0)
    c0 = _edges(ett, eft, wt0, wf0, b0)
    c1 = _edges(ett, eft, wt1, wf1, b1)
    h1p = _sc_segment_pass(a0, b0t, c0, src, dst)
    a1, b1t, p1 = _nodes1(h1p, ws1, wd1)
    p2r = _sc_sum_pass(a1, b1t, c1, src, dst)
    return _final(p2r, p0, p1, esum, wp0, wp1, wp2, bp)
